# baseline (speedup 1.0000x reference)
import functools

import jax
import jax.numpy as jnp
from jax import lax
from jax.experimental import pallas as pl
from jax.experimental.pallas import tpu as pltpu
from jax.experimental.pallas import tpu_sc as plsc

_N = 10000
_E = 320000
_D = 128
_ALPHA = 0.9

_NC = 2
_NS = 16
_NW = _NC * _NS
_FPT = _D // _NW
_NP = 10240
_EC = 4096
_NCH = 80
_E_PAD = _NCH * _EC


def _sc_body(xt_hbm, src_hbm, dst_hbm, val_hbm, out_hbm,
             xloc, aggloc, sb0, db0, vb0, sb1, db1, vb1, sem0, sem1):
    c = lax.axis_index("c")
    s = lax.axis_index("s")
    w = c * _NS + s
    fb = w * (_FPT * _NP)

    sbuf = (sb0, sb1)
    dbuf = (db0, db1)
    vbuf = (vb0, vb1)
    sem = (sem0, sem1)

    pltpu.sync_copy(xt_hbm.at[pl.ds(fb, _FPT * _NP)], xloc)

    z = jnp.zeros((16,), jnp.float32)

    def zset(i, carry):
        aggloc[pl.ds(i * 16, 16)] = z
        return carry

    lax.fori_loop(0, _FPT * _NP // 16, zset, 0)

    def stage_issue(ch, t):
        es = pl.ds(ch * _EC, _EC)
        pltpu.async_copy(src_hbm.at[es], sbuf[t], sem[t])
        pltpu.async_copy(dst_hbm.at[es], dbuf[t], sem[t])
        pltpu.async_copy(val_hbm.at[es], vbuf[t], sem[t])

    def stage_wait(ch, t):
        es = pl.ds(ch * _EC, _EC)
        pltpu.make_async_copy(src_hbm.at[es], sbuf[t], sem[t]).wait()
        pltpu.make_async_copy(dst_hbm.at[es], dbuf[t], sem[t]).wait()
        pltpu.make_async_copy(val_hbm.at[es], vbuf[t], sem[t]).wait()

    def process(t):
        @plsc.parallel_loop(0, _EC // 16, unroll=8)
        def group(g):
            gs = pl.ds(g * 16, 16)
            s16 = sbuf[t][gs]
            d16 = dbuf[t][gs]
            v16 = vbuf[t][gs]
            for k in range(_FPT):
                xv = plsc.load_gather(xloc, [s16 + (k * _NP)])
                plsc.addupdate_scatter(aggloc, [d16 + (k * _NP)], xv * v16)

    def step(ch, t, prefetch):
        if prefetch:
            stage_issue(ch + 1, 1 - t)
        stage_wait(ch, t)
        process(t)

    stage_issue(0, 0)

    def pair(m, carry):
        ch0 = 2 * m
        step(ch0, 0, True)
        step(ch0 + 1, 1, True)
        return carry

    lax.fori_loop(0, _NCH // 2 - 1, pair, 0)
    step(_NCH - 2, 0, True)
    step(_NCH - 1, 1, False)

    pltpu.sync_copy(aggloc, out_hbm.at[pl.ds(fb, _FPT * _NP)])


_sc_gather_scatter = functools.partial(
    pl.kernel,
    out_type=jax.ShapeDtypeStruct((_D * _NP,), jnp.float32),
    mesh=plsc.VectorSubcoreMesh(core_axis_name="c", subcore_axis_name="s"),
    compiler_params=pltpu.CompilerParams(needs_layout_passes=False),
    scratch_types=[
        pltpu.VMEM((_FPT * _NP,), jnp.float32),
        pltpu.VMEM((_FPT * _NP,), jnp.float32),
        pltpu.VMEM((_EC,), jnp.int32),
        pltpu.VMEM((_EC,), jnp.int32),
        pltpu.VMEM((_EC,), jnp.float32),
        pltpu.VMEM((_EC,), jnp.int32),
        pltpu.VMEM((_EC,), jnp.int32),
        pltpu.VMEM((_EC,), jnp.float32),
        pltpu.SemaphoreType.DMA,
        pltpu.SemaphoreType.DMA,
    ],
)(_sc_body)


_BLK = 512


def _tc_body(pT_ref, xi_ref, wi_ref, wx_ref, o_ref):
    agg = jnp.transpose(pT_ref[...])
    h = _ALPHA * agg + (1.0 - _ALPHA) * jnp.dot(
        xi_ref[...], wi_ref[...], preferred_element_type=jnp.float32)
    o_ref[...] = jnp.maximum(
        jnp.dot(h, wx_ref[...], preferred_element_type=jnp.float32), 0.0)


def _tc_dense(aggT, xi, wi, wx):
    nblk = _NP // _BLK
    return pl.pallas_call(
        _tc_body,
        out_shape=jax.ShapeDtypeStruct((_NP, _D), jnp.float32),
        grid=(nblk,),
        in_specs=[
            pl.BlockSpec((_D, _BLK), lambda i: (0, i)),
            pl.BlockSpec((_BLK, 8), lambda i: (i, 0)),
            pl.BlockSpec((8, _D), lambda i: (0, 0)),
            pl.BlockSpec((_D, _D), lambda i: (0, 0)),
        ],
        out_specs=pl.BlockSpec((_BLK, _D), lambda i: (i, 0)),
    )(aggT, xi, wi, wx)


def kernel(x, x_initial, edge_index, adj_values, w_init, w_x):
    dst = edge_index[0]
    src = edge_index[1]
    pad = _E_PAD - _E
    zi = jnp.zeros((pad,), jnp.int32)
    srcp = jnp.concatenate([src, zi])
    dstp = jnp.concatenate([dst, zi])
    valp = jnp.concatenate([adj_values, jnp.zeros((pad,), jnp.float32)])

    xt = jnp.pad(x.T, ((0, 0), (0, _NP - _N))).reshape(_D * _NP)

    aggT = _sc_gather_scatter(xt, srcp, dstp, valp).reshape(_D, _NP)

    xi = jnp.pad(x_initial, ((0, _NP - _N), (0, 5)))
    wi = jnp.pad(w_init, ((0, 5), (0, 0)))
    return _tc_dense(aggT, xi, wi, w_x)[:_N]


# speedup vs baseline: 1.0215x; 1.0215x over previous
import functools

import jax
import jax.numpy as jnp
from jax import lax
from jax.experimental import pallas as pl
from jax.experimental.pallas import tpu as pltpu
from jax.experimental.pallas import tpu_sc as plsc

_N = 10000
_E = 320000
_D = 128
_ALPHA = 0.9

_NC = 2
_NS = 16
_NW = _NC * _NS
_FPT = _D // _NW
_NP = 10240
_EC = 4096
_NCH = 80
_E_PAD = _NCH * _EC


def _sc_body(xt_hbm, src_hbm, dst_hbm, val_hbm, out_hbm,
             xloc, aggloc, sb0, db0, vb0, sb1, db1, vb1, sem0, sem1):
    c = lax.axis_index("c")
    s = lax.axis_index("s")
    w = c * _NS + s
    fb = w * (_FPT * _NP)

    sbuf = (sb0, sb1)
    dbuf = (db0, db1)
    vbuf = (vb0, vb1)
    sem = (sem0, sem1)

    pltpu.sync_copy(xt_hbm.at[pl.ds(fb, _FPT * _NP)], xloc)

    z = jnp.zeros((16,), jnp.float32)

    def zset(i, carry):
        aggloc[pl.ds(i * 16, 16)] = z
        return carry

    lax.fori_loop(0, _FPT * _NP // 16, zset, 0)

    def stage_issue(ch, t):
        es = pl.ds(ch * _EC, _EC)
        pltpu.async_copy(src_hbm.at[es], sbuf[t], sem[t])
        pltpu.async_copy(dst_hbm.at[es], dbuf[t], sem[t])
        pltpu.async_copy(val_hbm.at[es], vbuf[t], sem[t])

    def stage_wait(ch, t):
        es = pl.ds(ch * _EC, _EC)
        pltpu.make_async_copy(src_hbm.at[es], sbuf[t], sem[t]).wait()
        pltpu.make_async_copy(dst_hbm.at[es], dbuf[t], sem[t]).wait()
        pltpu.make_async_copy(val_hbm.at[es], vbuf[t], sem[t]).wait()

    def process(t):
        @plsc.parallel_loop(0, _EC // 16, unroll=4)
        def group(g):
            gs = pl.ds(g * 16, 16)
            s16 = sbuf[t][gs]
            d16 = dbuf[t][gs]
            v16 = vbuf[t][gs]
            for k in range(_FPT):
                xv = plsc.load_gather(xloc, [s16 + (k * _NP)])
                plsc.addupdate_scatter(aggloc, [d16 + (k * _NP)], xv * v16)

    def step(ch, t, prefetch):
        if prefetch:
            stage_issue(ch + 1, 1 - t)
        stage_wait(ch, t)
        process(t)

    stage_issue(0, 0)

    def pair(m, carry):
        ch0 = 2 * m
        step(ch0, 0, True)
        step(ch0 + 1, 1, True)
        return carry

    lax.fori_loop(0, _NCH // 2 - 1, pair, 0)
    step(_NCH - 2, 0, True)
    step(_NCH - 1, 1, False)

    pltpu.sync_copy(aggloc, out_hbm.at[pl.ds(fb, _FPT * _NP)])


_sc_gather_scatter = functools.partial(
    pl.kernel,
    out_type=jax.ShapeDtypeStruct((_D * _NP,), jnp.float32),
    mesh=plsc.VectorSubcoreMesh(core_axis_name="c", subcore_axis_name="s"),
    compiler_params=pltpu.CompilerParams(needs_layout_passes=False),
    scratch_types=[
        pltpu.VMEM((_FPT * _NP,), jnp.float32),
        pltpu.VMEM((_FPT * _NP,), jnp.float32),
        pltpu.VMEM((_EC,), jnp.int32),
        pltpu.VMEM((_EC,), jnp.int32),
        pltpu.VMEM((_EC,), jnp.float32),
        pltpu.VMEM((_EC,), jnp.int32),
        pltpu.VMEM((_EC,), jnp.int32),
        pltpu.VMEM((_EC,), jnp.float32),
        pltpu.SemaphoreType.DMA,
        pltpu.SemaphoreType.DMA,
    ],
)(_sc_body)


_BLK = 512


def _tc_body(pT_ref, xi_ref, wi_ref, wx_ref, o_ref):
    agg = jnp.transpose(pT_ref[...])
    h = _ALPHA * agg + (1.0 - _ALPHA) * jnp.dot(
        xi_ref[...], wi_ref[...], preferred_element_type=jnp.float32)
    o_ref[...] = jnp.maximum(
        jnp.dot(h, wx_ref[...], preferred_element_type=jnp.float32), 0.0)


def _tc_dense(aggT, xi, wi, wx):
    nblk = _NP // _BLK
    return pl.pallas_call(
        _tc_body,
        out_shape=jax.ShapeDtypeStruct((_NP, _D), jnp.float32),
        grid=(nblk,),
        in_specs=[
            pl.BlockSpec((_D, _BLK), lambda i: (0, i)),
            pl.BlockSpec((_BLK, 8), lambda i: (i, 0)),
            pl.BlockSpec((8, _D), lambda i: (0, 0)),
            pl.BlockSpec((_D, _D), lambda i: (0, 0)),
        ],
        out_specs=pl.BlockSpec((_BLK, _D), lambda i: (i, 0)),
    )(aggT, xi, wi, wx)


def kernel(x, x_initial, edge_index, adj_values, w_init, w_x):
    dst = edge_index[0]
    src = edge_index[1]
    pad = _E_PAD - _E
    zi = jnp.zeros((pad,), jnp.int32)
    srcp = jnp.concatenate([src, zi])
    dstp = jnp.concatenate([dst, zi])
    valp = jnp.concatenate([adj_values, jnp.zeros((pad,), jnp.float32)])

    xt = jnp.pad(x.T, ((0, 0), (0, _NP - _N))).reshape(_D * _NP)

    aggT = _sc_gather_scatter(xt, srcp, dstp, valp).reshape(_D, _NP)

    xi = jnp.pad(x_initial, ((0, _NP - _N), (0, 5)))
    wi = jnp.pad(w_init, ((0, 5), (0, 0)))
    return _tc_dense(aggT, xi, wi, w_x)[:_N]
